# baseline (speedup 1.0000x reference)
import functools

import jax
import jax.numpy as jnp
from jax import lax
from jax.experimental import pallas as pl
from jax.experimental.pallas import tpu as pltpu
from jax.experimental.pallas import tpu_sc as plsc

B = 16384
D = 64
NC = 2
NS = 16
NW = NC * NS
B_PER_W = B // NW
FIRE = 16
N_BATCH = B_PER_W // FIRE

BLK = 2048
GRID = B // BLK


def _sc_gather_body(uid_hbm, mid_hbm, uid_tabT, mid_tab, uid_out, mid_out,
                    uidx_v, midx_v, rows_v,
                    bin0_v, bin1_v, bin2_v, bin3_v, bin4_v, bin5_v,
                    msem, bsem0, bsem1, bsem2, bsem3, bsem4, bsem5):
    wid = lax.axis_index("s") * NC + lax.axis_index("c")
    base = wid * B_PER_W
    bufs = (bin0_v, bin1_v, bin2_v, bin3_v, bin4_v, bin5_v)
    bsems = (bsem0, bsem1, bsem2, bsem3, bsem4, bsem5)

    pltpu.sync_copy(uid_hbm.at[wid], uidx_v)
    pltpu.sync_copy(mid_hbm.at[wid], midx_v)

    def _fire(u, binbuf, bsem):
        off = pl.multiple_of((u >> 7) << 7, 128)
        return pltpu.async_copy(
            uid_tabT.at[:, pl.ds(off, 128)], binbuf, bsem)

    def _extract(i, u, binbuf):
        rmv = jnp.full((16,), u & 127, jnp.int32)
        for t in range(4):
            pos = lax.iota(jnp.int32, 16) + t * 16
            rows_v[i, pl.ds(t * 16, 16)] = plsc.load_gather(binbuf, [pos, rmv])

    def batch(c, _):
        mvec = midx_v[pl.ds(c * 16, 16)]
        mcps = []
        for k in range(16):
            r = mvec[k]
            mcps.append(pltpu.async_copy(
                mid_tab.at[pl.ds(r, 1)],
                mid_out.at[pl.ds(base + c * 16 + k, 1)], msem))

        vec = uidx_v[pl.ds(c * 16, 16)]
        cps = [_fire(vec[k], bufs[k], bsems[k]) for k in range(6)]
        for k in range(10):
            cps[k % 6].wait()
            _extract(c * 16 + k, vec[k], bufs[k % 6])
            cps[k % 6] = _fire(vec[6 + k], bufs[k % 6], bsems[k % 6])
        for k in range(10, 16):
            cps[k % 6].wait()
            _extract(c * 16 + k, vec[k], bufs[k % 6])

        for cp in mcps:
            cp.wait()
        return 0

    lax.fori_loop(0, B_PER_W // 16, batch, 0)
    pltpu.sync_copy(rows_v, uid_out.at[pl.ds(base, B_PER_W)])


@functools.cache
def _sc_gather():
    return pl.kernel(
        _sc_gather_body,
        out_type=(jax.ShapeDtypeStruct((B, D), jnp.float32),
                  jax.ShapeDtypeStruct((B, D), jnp.float32)),
        mesh=plsc.VectorSubcoreMesh(core_axis_name="c", subcore_axis_name="s"),
        scratch_types=[
            pltpu.VMEM((B_PER_W,), jnp.int32),
            pltpu.VMEM((B_PER_W,), jnp.int32),
            pltpu.VMEM((B_PER_W, D), jnp.float32),
            pltpu.VMEM((D, 128), jnp.float32),
            pltpu.VMEM((D, 128), jnp.float32),
            pltpu.VMEM((D, 128), jnp.float32),
            pltpu.VMEM((D, 128), jnp.float32),
            pltpu.VMEM((D, 128), jnp.float32),
            pltpu.VMEM((D, 128), jnp.float32),
            pltpu.SemaphoreType.DMA,
            pltpu.SemaphoreType.DMA,
            pltpu.SemaphoreType.DMA,
            pltpu.SemaphoreType.DMA,
            pltpu.SemaphoreType.DMA,
            pltpu.SemaphoreType.DMA,
            pltpu.SemaphoreType.DMA,
        ],
        compiler_params=pltpu.CompilerParams(needs_layout_passes=False),
    )


def _tc_body(g_ref, a_ref, j_ref, eu_ref, em_ref,
             eg_tab, ea_tab, ej_tab, w_ref, b_ref, o_ref):
    em = em_ref[...]
    eu = eu_ref[...]
    w_top = w_ref[0:D, :]
    w_bot = w_ref[D:2 * D, :]
    m = jnp.dot(em, w_top, preferred_element_type=jnp.float32) + b_ref[...]
    acc_t = jnp.zeros((BLK, D), jnp.float32)
    acc_m = jnp.zeros((BLK, D), jnp.float32)
    for idx_ref, tab_ref, n in ((g_ref, eg_tab, 2),
                                (a_ref, ea_tab, 7),
                                (j_ref, ej_tab, 21)):
        tab = tab_ref[...]
        oh = (idx_ref[...] ==
              lax.broadcasted_iota(jnp.int32, (BLK, n), 1)).astype(jnp.float32)
        e_attr = jnp.dot(oh, tab, preferred_element_type=jnp.float32)
        tab_w = jnp.dot(tab, w_bot, preferred_element_type=jnp.float32)
        v = m + jnp.dot(oh, tab_w, preferred_element_type=jnp.float32)
        v = v - jnp.max(v, axis=1, keepdims=True)
        ev = jnp.exp(v)
        wgt = (ev / jnp.sum(ev, axis=1, keepdims=True)) * e_attr
        acc_t = acc_t + wgt
        acc_m = acc_m + wgt * wgt
    p = eu * acc_t + 0.5 * (acc_t * acc_t - acc_m)
    o_ref[...] = jnp.sum(p * em, axis=1, keepdims=True)


_tc_forward = pl.pallas_call(
    _tc_body,
    grid=(GRID,),
    in_specs=[
        pl.BlockSpec((BLK, 1), lambda i: (i, 0)),
        pl.BlockSpec((BLK, 1), lambda i: (i, 0)),
        pl.BlockSpec((BLK, 1), lambda i: (i, 0)),
        pl.BlockSpec((BLK, D), lambda i: (i, 0)),
        pl.BlockSpec((BLK, D), lambda i: (i, 0)),
        pl.BlockSpec((2, D), lambda i: (0, 0)),
        pl.BlockSpec((7, D), lambda i: (0, 0)),
        pl.BlockSpec((21, D), lambda i: (0, 0)),
        pl.BlockSpec((2 * D, D), lambda i: (0, 0)),
        pl.BlockSpec((1, D), lambda i: (0, 0)),
    ],
    out_specs=pl.BlockSpec((BLK, 1), lambda i: (i, 0)),
    out_shape=jax.ShapeDtypeStruct((B, 1), jnp.float32),
)


def kernel(uid, gender, age, job, mid, E_uid, E_gender, E_age, E_job, E_mid,
           att_W, att_b):
    uid2 = uid.reshape(NW, B_PER_W)
    mid2 = mid.reshape(NW, B_PER_W)
    e_uid, e_mid = _sc_gather()(uid2, mid2, E_uid.T, E_mid)
    return _tc_forward(gender.reshape(B, 1), age.reshape(B, 1),
                       job.reshape(B, 1), e_uid, e_mid,
                       E_gender, E_age, E_job, att_W, att_b.reshape(1, D))


# speedup vs baseline: 1.0345x; 1.0345x over previous
import functools

import jax
import jax.numpy as jnp
from jax import lax
from jax.experimental import pallas as pl
from jax.experimental.pallas import tpu as pltpu
from jax.experimental.pallas import tpu_sc as plsc

B = 16384
D = 64
NC = 2
NS = 16
NW = NC * NS
N_CHUNK = 2
B_CHUNK = B // N_CHUNK
B_PER_W = B_CHUNK // NW

BLK = 2048
GRID = B_CHUNK // BLK


def _sc_gather_body(uid_hbm, mid_hbm, uid_tabT, mid_tab, uid_out, mid_out,
                    uidx_v, midx_v, rows_v,
                    bin0_v, bin1_v, bin2_v, bin3_v, bin4_v, bin5_v,
                    msem, bsem0, bsem1, bsem2, bsem3, bsem4, bsem5):
    wid = lax.axis_index("s") * NC + lax.axis_index("c")
    base = wid * B_PER_W
    bufs = (bin0_v, bin1_v, bin2_v, bin3_v, bin4_v, bin5_v)
    bsems = (bsem0, bsem1, bsem2, bsem3, bsem4, bsem5)

    pltpu.sync_copy(uid_hbm.at[wid], uidx_v)
    pltpu.sync_copy(mid_hbm.at[wid], midx_v)

    def _fire(u, binbuf, bsem):
        off = pl.multiple_of((u >> 7) << 7, 128)
        return pltpu.async_copy(
            uid_tabT.at[:, pl.ds(off, 128)], binbuf, bsem)

    def _extract(i, u, binbuf):
        rmv = jnp.full((16,), u & 127, jnp.int32)
        for t in range(4):
            pos = lax.iota(jnp.int32, 16) + t * 16
            rows_v[i, pl.ds(t * 16, 16)] = plsc.load_gather(binbuf, [pos, rmv])

    def batch(c, _):
        mvec = midx_v[pl.ds(c * 16, 16)]
        mcps = []
        for k in range(16):
            r = mvec[k]
            mcps.append(pltpu.async_copy(
                mid_tab.at[pl.ds(r, 1)],
                mid_out.at[pl.ds(base + c * 16 + k, 1)], msem))

        vec = uidx_v[pl.ds(c * 16, 16)]
        cps = [_fire(vec[k], bufs[k], bsems[k]) for k in range(6)]
        for k in range(10):
            cps[k % 6].wait()
            _extract(c * 16 + k, vec[k], bufs[k % 6])
            cps[k % 6] = _fire(vec[6 + k], bufs[k % 6], bsems[k % 6])
        for k in range(10, 16):
            cps[k % 6].wait()
            _extract(c * 16 + k, vec[k], bufs[k % 6])

        for cp in mcps:
            cp.wait()
        return 0

    lax.fori_loop(0, B_PER_W // 16, batch, 0)
    pltpu.sync_copy(rows_v, uid_out.at[pl.ds(base, B_PER_W)])


@functools.cache
def _sc_gather():
    return pl.kernel(
        _sc_gather_body,
        out_type=(jax.ShapeDtypeStruct((B_CHUNK, D), jnp.float32),
                  jax.ShapeDtypeStruct((B_CHUNK, D), jnp.float32)),
        mesh=plsc.VectorSubcoreMesh(core_axis_name="c", subcore_axis_name="s"),
        scratch_types=[
            pltpu.VMEM((B_PER_W,), jnp.int32),
            pltpu.VMEM((B_PER_W,), jnp.int32),
            pltpu.VMEM((B_PER_W, D), jnp.float32),
            pltpu.VMEM((D, 128), jnp.float32),
            pltpu.VMEM((D, 128), jnp.float32),
            pltpu.VMEM((D, 128), jnp.float32),
            pltpu.VMEM((D, 128), jnp.float32),
            pltpu.VMEM((D, 128), jnp.float32),
            pltpu.VMEM((D, 128), jnp.float32),
            pltpu.SemaphoreType.DMA,
            pltpu.SemaphoreType.DMA,
            pltpu.SemaphoreType.DMA,
            pltpu.SemaphoreType.DMA,
            pltpu.SemaphoreType.DMA,
            pltpu.SemaphoreType.DMA,
            pltpu.SemaphoreType.DMA,
        ],
        compiler_params=pltpu.CompilerParams(needs_layout_passes=False),
    )


def _tc_body(g_ref, a_ref, j_ref, eu_ref, em_ref,
             eg_tab, ea_tab, ej_tab, w_ref, b_ref, o_ref):
    em = em_ref[...]
    eu = eu_ref[...]
    w_top = w_ref[0:D, :]
    w_bot = w_ref[D:2 * D, :]
    m = jnp.dot(em, w_top, preferred_element_type=jnp.float32) + b_ref[...]
    acc_t = jnp.zeros((BLK, D), jnp.float32)
    acc_m = jnp.zeros((BLK, D), jnp.float32)
    for idx_ref, tab_ref, n in ((g_ref, eg_tab, 2),
                                (a_ref, ea_tab, 7),
                                (j_ref, ej_tab, 21)):
        tab = tab_ref[...]
        oh = (idx_ref[...] ==
              lax.broadcasted_iota(jnp.int32, (BLK, n), 1)).astype(jnp.float32)
        e_attr = jnp.dot(oh, tab, preferred_element_type=jnp.float32)
        tab_w = jnp.dot(tab, w_bot, preferred_element_type=jnp.float32)
        v = m + jnp.dot(oh, tab_w, preferred_element_type=jnp.float32)
        v = v - jnp.max(v, axis=1, keepdims=True)
        ev = jnp.exp(v)
        wgt = (ev / jnp.sum(ev, axis=1, keepdims=True)) * e_attr
        acc_t = acc_t + wgt
        acc_m = acc_m + wgt * wgt
    p = eu * acc_t + 0.5 * (acc_t * acc_t - acc_m)
    o_ref[...] = jnp.sum(p * em, axis=1, keepdims=True)


_tc_forward = pl.pallas_call(
    _tc_body,
    grid=(GRID,),
    in_specs=[
        pl.BlockSpec((BLK, 1), lambda i: (i, 0)),
        pl.BlockSpec((BLK, 1), lambda i: (i, 0)),
        pl.BlockSpec((BLK, 1), lambda i: (i, 0)),
        pl.BlockSpec((BLK, D), lambda i: (i, 0)),
        pl.BlockSpec((BLK, D), lambda i: (i, 0)),
        pl.BlockSpec((2, D), lambda i: (0, 0)),
        pl.BlockSpec((7, D), lambda i: (0, 0)),
        pl.BlockSpec((21, D), lambda i: (0, 0)),
        pl.BlockSpec((2 * D, D), lambda i: (0, 0)),
        pl.BlockSpec((1, D), lambda i: (0, 0)),
    ],
    out_specs=pl.BlockSpec((BLK, 1), lambda i: (i, 0)),
    out_shape=jax.ShapeDtypeStruct((B_CHUNK, 1), jnp.float32),
)


def kernel(uid, gender, age, job, mid, E_uid, E_gender, E_age, E_job, E_mid,
           att_W, att_b):
    scg = _sc_gather()
    E_uidT = E_uid.T
    att_b2 = att_b.reshape(1, D)
    outs = []
    for t in range(N_CHUNK):
        sl = slice(t * B_CHUNK, (t + 1) * B_CHUNK)
        e_uid, e_mid = scg(uid[sl].reshape(NW, B_PER_W),
                           mid[sl].reshape(NW, B_PER_W), E_uidT, E_mid)
        outs.append(_tc_forward(gender[sl].reshape(B_CHUNK, 1),
                                age[sl].reshape(B_CHUNK, 1),
                                job[sl].reshape(B_CHUNK, 1), e_uid, e_mid,
                                E_gender, E_age, E_job, att_W, att_b2))
    return jnp.concatenate(outs, axis=0)
